# baseline (speedup 1.0000x reference)
import functools

import jax
import jax.numpy as jnp
from jax import lax
from jax.experimental import pallas as pl
from jax.experimental.pallas import tpu as pltpu
from jax.experimental.pallas import tpu_sc as plsc

N = 10000
E = 320000
F = 64
NC, NS, L = 2, 16, 16
NW = NC * NS
E_W = E // NW
CH = 80
NCHUNK = E_W // CH
RW = 80
RZ_STRIDE = 624
RZ_SIZE = 640
GR = 8


def _leaky(x):
    return jnp.maximum(x, 0.01 * x)


def _elu(x):
    return jnp.where(x > 0, x, jnp.exp(jnp.minimum(x, 0.0)) - 1.0)


def _gru_tc(ctx, h, wih_t, whh_t, bih, bhh):
    gi = jnp.dot(ctx, wih_t) + bih
    gh = jnp.dot(h, whh_t) + bhh
    r = jax.nn.sigmoid(gi[:, 0:F] + gh[:, 0:F])
    z = jax.nn.sigmoid(gi[:, F:2 * F] + gh[:, F:2 * F])
    nn_ = jnp.tanh(gi[:, 2 * F:] + r * gh[:, 2 * F:])
    return (1.0 - z) * nn_ + z * h


def _treesum(vs):
    while len(vs) > 1:
        vs = [a + b for a, b in zip(vs[0::2], vs[1::2])] + (
            [vs[-1]] if len(vs) % 2 else [])
    return vs[0]


def _tc_prep_body(nf_ref, pnw_ref, pnb_ref, w1a_ref, b1_ref, w2a_ref, b2_ref,
                  hv_ref, nfa_ref, d1_ref):
    nf = nf_ref[...]
    hv = _leaky(jnp.dot(nf, pnw_ref[...]) + pnb_ref[...])
    hv_ref[...] = hv
    nfa_ref[...] = jnp.dot(nf, w1a_ref[...]) + b1_ref[...]
    d1_ref[...] = jnp.dot(hv, w2a_ref[...]) + b2_ref[...]


def _tc_prep(nf, pnw, pnb, w1a, b1, w2a, b2):
    return pl.pallas_call(
        _tc_prep_body,
        out_shape=(
            jax.ShapeDtypeStruct((N, F), jnp.float32),
            jax.ShapeDtypeStruct((N, F), jnp.float32),
            jax.ShapeDtypeStruct((N, 1), jnp.float32),
        ),
    )(nf, pnw, pnb, w1a, b1, w2a, b2)


def _tc_mid_body(part_ref, hv_ref, g1w_ref, g1b_ref, wih_ref, whh_ref,
                 bih_ref, bhh_ref, wd_ref, pb_ref, ws_ref,
                 h_ref, dl_ref, sl_ref):
    pa = part_ref[0]
    pb2 = part_ref[1]
    s = pa[:, F:F + 1] + pb2[:, F:F + 1]
    wsum = pa[:, 0:F] + pb2[:, 0:F]
    pos = s > 0
    inv = jnp.where(pos, 1.0 / jnp.where(pos, s, 1.0), 0.0)
    ind = pos.astype(jnp.float32)
    ctx = _elu(jnp.dot(wsum * inv, g1w_ref[...]) + ind * g1b_ref[...])
    hv = hv_ref[...]
    h = jax.nn.relu(_gru_tc(ctx, hv, wih_ref[...], whh_ref[...],
                            bih_ref[...], bhh_ref[...]))
    h_ref[...] = h
    dl_ref[...] = jnp.dot(h, wd_ref[...]) + pb_ref[...]
    sl_ref[...] = jnp.dot(h, ws_ref[...])


def _tc_mid(part1, hv, g1w, g1b, wih, whh, bih, bhh, wd, pb, ws):
    return pl.pallas_call(
        _tc_mid_body,
        out_shape=(
            jax.ShapeDtypeStruct((N, F), jnp.float32),
            jax.ShapeDtypeStruct((N, 1), jnp.float32),
            jax.ShapeDtypeStruct((N, 1), jnp.float32),
        ),
    )(part1, hv, g1w, g1b, wih, whh, bih, bhh, wd, pb, ws)


def _tc_final_body(part_ref, h_ref, pnw_ref, pnb_ref, wih_ref, whh_ref,
                   bih_ref, bhh_ref,
                   cg0_ref, ch0_ref, cb0_ref, rpw0_ref, rpb0_ref,
                   rwih0_ref, rwhh0_ref, rbih0_ref, rbhh0_ref,
                   cg1_ref, ch1_ref, cb1_ref, rpw1_ref, rpb1_ref,
                   rwih1_ref, rwhh1_ref, rbih1_ref, rbhh1_ref,
                   g_ref):
    pa = part_ref[0]
    pb2 = part_ref[1]
    s = pa[:, F:F + 1] + pb2[:, F:F + 1]
    wsum = pa[:, 0:F] + pb2[:, 0:F]
    pos = s > 0
    inv = jnp.where(pos, 1.0 / jnp.where(pos, s, 1.0), 0.0)
    ind = pos.astype(jnp.float32)
    ctx = _elu(jnp.dot(wsum * inv, pnw_ref[...]) + ind * pnb_ref[...])
    h_prev = h_ref[...]
    h = jax.nn.relu(_gru_tc(ctx, h_prev, wih_ref[...], whh_ref[...],
                            bih_ref[...], bhh_ref[...]))
    g = jnp.sum(h, axis=0, keepdims=True)
    ro = [(cg0_ref, ch0_ref, cb0_ref, rpw0_ref, rpb0_ref,
           rwih0_ref, rwhh0_ref, rbih0_ref, rbhh0_ref),
          (cg1_ref, ch1_ref, cb1_ref, rpw1_ref, rpb1_ref,
           rwih1_ref, rwhh1_ref, rbih1_ref, rbhh1_ref)]
    for (cg, chh, cb, rpw, rpb, rwih, rwhh, rbih, rbhh) in ro:
        zc = jnp.dot(jax.nn.relu(g), cg[...]) + cb[...]
        z = _leaky(zc + jnp.dot(h, chh[...]))
        zm = jnp.max(z, axis=0, keepdims=True)
        ez = jnp.exp(z - zm)
        aa = ez / jnp.sum(ez, axis=0, keepdims=True)
        hvp = jnp.dot(h, rpw[...]) + rpb[...]
        g_repr = _elu(jnp.sum(aa * hvp, axis=0, keepdims=True))
        g = _gru_tc(jax.nn.relu(g_repr), g, rwih[...], rwhh[...],
                    rbih[...], rbhh[...])
    g_ref[...] = g


def _tc_final(part2, h, args):
    return pl.pallas_call(
        _tc_final_body,
        out_shape=jax.ShapeDtypeStruct((1, F), jnp.float32),
    )(part2, h, *args)


def _sc_round1(src3, dst3, nfa, ef3, d1, w2b, w1b, zeros):
    mesh = plsc.VectorSubcoreMesh(core_axis_name="c", subcore_axis_name="s",
                                  num_cores=NC, num_subcores=NS)

    @functools.partial(
        pl.kernel,
        out_type=jax.ShapeDtypeStruct((NC, N, RW), jnp.float32),
        mesh=mesh,
        compiler_params=pltpu.CompilerParams(needs_layout_passes=False,
                                             use_tc_tiling_on_sc=False),
        scratch_types=[
            pltpu.VMEM((NCHUNK, CH), jnp.int32),
            pltpu.VMEM((NCHUNK, CH), jnp.int32),
            pltpu.VMEM((CH, F), jnp.float32),
            pltpu.VMEM((CH, F), jnp.float32),
            pltpu.VMEM((CH * 4,), jnp.float32),
            pltpu.VMEM((CH * 4,), jnp.float32),
            pltpu.VMEM((CH, RW), jnp.float32),
            pltpu.VMEM((CH, RW), jnp.float32),
            pltpu.VMEM((L, L), jnp.float32),
            pltpu.VMEM((N,), jnp.float32),
            pltpu.VMEM((F,), jnp.float32),
            pltpu.VMEM((4 * F,), jnp.float32),
            pltpu.VMEM_SHARED((N, RW), jnp.float32),
            pltpu.SemaphoreType.DMA,
            pltpu.SemaphoreType.DMA,
            pltpu.SemaphoreType.DMA,
            pltpu.SemaphoreType.DMA,
            pltpu.SemaphoreType.DMA,
            pltpu.SemaphoreType.DMA,
        ],
    )
    def body(src_hbm, dst_hbm, nfa_hbm, ef_hbm, d1_hbm, w2b_hbm, w1b_hbm,
             z_hbm, out_hbm, srcv, dstv, ga, gb, ea, eb, rows_a, rows_b,
             accv, d1_v, w2b_v, w1b_v, table, sga, sgb, sea, seb, ssa, ssb):
        cid = lax.axis_index("c")
        sid = lax.axis_index("s")
        wid = sid * NC + cid
        pltpu.sync_copy(d1_hbm, d1_v)
        pltpu.sync_copy(w2b_hbm, w2b_v)
        pltpu.sync_copy(w1b_hbm, w1b_v)
        pltpu.sync_copy(src_hbm.at[wid], srcv)
        pltpu.sync_copy(dst_hbm.at[wid], dstv)
        pltpu.sync_copy(z_hbm.at[pl.ds(sid * RZ_STRIDE, RZ_SIZE)],
                        table.at[pl.ds(sid * RZ_STRIDE, RZ_SIZE)])
        plsc.subcore_barrier()
        w2 = [w2b_v[pl.ds(c * L, L)] for c in range(4)]
        w1 = [[w1b_v[pl.ds(k * F + c * L, L)] for c in range(4)]
              for k in range(4)]
        iota = lax.iota(jnp.int32, L)

        def prime(ci, gbuf, ebuf, sg, se):
            pltpu.async_copy(nfa_hbm.at[srcv.at[ci]], gbuf, sg)
            pltpu.async_copy(ef_hbm.at[wid, ci], ebuf, se)

        def waitbuf(gbuf, ebuf, sg, se):
            pltpu.make_async_copy(nfa_hbm.at[pl.ds(0, CH)], gbuf, sg).wait()
            pltpu.make_async_copy(ef_hbm.at[0, 0], ebuf, se).wait()

        def compute(ci, gbuf, ebuf, rows, ssem):
            pltpu.make_async_copy(rows, table.at[dstv.at[ci]], ssem).wait()
            civ = jnp.full((L,), ci, jnp.int32)

            def group_body(g, c2):
                e0 = g * L
                dst16 = plsc.load_gather(dstv, [civ, iota + e0])
                d1g = plsc.load_gather(d1_v, [dst16])
                for k in range(L):
                    e = e0 + k
                    efs = [plsc.load_gather(
                        ebuf, [jnp.full((L,), e * 4 + kk, jnp.int32)])
                        for kk in range(4)]
                    ts = []
                    for c in range(4):
                        ep = ((efs[0] * w1[0][c] + efs[1] * w1[1][c])
                              + (efs[2] * w1[2][c] + efs[3] * w1[3][c]))
                        x = gbuf[e, pl.ds(c * L, L)] + ep
                        h1 = jnp.maximum(x, 0.01 * x)
                        rows[e, pl.ds(c * L, L)] = h1
                        ts.append(h1 * w2[c])
                    accv[k] = (ts[0] + ts[1]) + (ts[2] + ts[3])
                cols = [plsc.load_gather(accv,
                                         [iota, jnp.full((L,), l, jnp.int32)])
                        for l in range(L)]
                l0 = d1g + _treesum(cols)
                pvv = jnp.exp(jnp.maximum(l0, 0.01 * l0))
                for k in range(L):
                    e = e0 + k
                    pspl = jnp.full((L,), pvv[k], jnp.float32)
                    for c in range(4):
                        rows[e, pl.ds(c * L, L)] = (
                            rows[e, pl.ds(c * L, L)] * pspl)
                    rows[e, pl.ds(F, L)] = pspl
                return c2

            lax.fori_loop(0, CH // L, group_body, 0)
            pltpu.async_copy(rows, table.at[dstv.at[ci]], ssem, add=True)

        pltpu.sync_copy(z_hbm.at[pl.ds(0, CH)], rows_a)
        pltpu.sync_copy(z_hbm.at[pl.ds(0, CH)], rows_b)
        pltpu.async_copy(rows_a, table.at[dstv.at[0]], ssa, add=True)
        pltpu.async_copy(rows_b, table.at[dstv.at[0]], ssb, add=True)
        prime(0, ga, ea, sga, sea)

        def pair_body(t, carry):
            c0 = 2 * t
            prime(c0 + 1, gb, eb, sgb, seb)
            waitbuf(ga, ea, sga, sea)
            compute(c0, ga, ea, rows_a, ssa)
            prime(c0 + 2, ga, ea, sga, sea)
            waitbuf(gb, eb, sgb, seb)
            compute(c0 + 1, gb, eb, rows_b, ssb)
            return carry

        lax.fori_loop(0, (NCHUNK - 1) // 2, pair_body, 0)
        waitbuf(ga, ea, sga, sea)
        compute(NCHUNK - 1, ga, ea, rows_a, ssa)
        pltpu.make_async_copy(rows_a, table.at[dstv.at[0]], ssa).wait()
        pltpu.make_async_copy(rows_b, table.at[dstv.at[0]], ssb).wait()
        plsc.subcore_barrier()
        pltpu.sync_copy(table.at[pl.ds(sid * RZ_STRIDE, RZ_SIZE)],
                        out_hbm.at[cid, pl.ds(sid * RZ_STRIDE, RZ_SIZE)])

    return body(src3, dst3, nfa, ef3, d1, w2b, w1b, zeros)


def _sc_round2(src3, dst3, h, dl, sl, zeros):
    mesh = plsc.VectorSubcoreMesh(core_axis_name="c", subcore_axis_name="s",
                                  num_cores=NC, num_subcores=NS)

    @functools.partial(
        pl.kernel,
        out_type=jax.ShapeDtypeStruct((NC, N, RW), jnp.float32),
        mesh=mesh,
        compiler_params=pltpu.CompilerParams(needs_layout_passes=False,
                                             use_tc_tiling_on_sc=False),
        scratch_types=[
            pltpu.VMEM((NCHUNK, CH), jnp.int32),
            pltpu.VMEM((NCHUNK, CH), jnp.int32),
            pltpu.VMEM((CH, F), jnp.float32),
            pltpu.VMEM((CH, F), jnp.float32),
            pltpu.VMEM((CH, RW), jnp.float32),
            pltpu.VMEM((CH, RW), jnp.float32),
            pltpu.VMEM((N,), jnp.float32),
            pltpu.VMEM((N,), jnp.float32),
            pltpu.VMEM_SHARED((N, RW), jnp.float32),
            pltpu.SemaphoreType.DMA,
            pltpu.SemaphoreType.DMA,
            pltpu.SemaphoreType.DMA,
            pltpu.SemaphoreType.DMA,
        ],
    )
    def body(src_hbm, dst_hbm, h_hbm, dl_hbm, sl_hbm, z_hbm, out_hbm,
             srcv, dstv, ga, gb, rows_a, rows_b, dl_v, sl_v, table,
             sga, sgb, ssa, ssb):
        cid = lax.axis_index("c")
        sid = lax.axis_index("s")
        wid = sid * NC + cid
        pltpu.sync_copy(dl_hbm, dl_v)
        pltpu.sync_copy(sl_hbm, sl_v)
        pltpu.sync_copy(src_hbm.at[wid], srcv)
        pltpu.sync_copy(dst_hbm.at[wid], dstv)
        pltpu.sync_copy(z_hbm.at[pl.ds(sid * RZ_STRIDE, RZ_SIZE)],
                        table.at[pl.ds(sid * RZ_STRIDE, RZ_SIZE)])
        plsc.subcore_barrier()
        iota = lax.iota(jnp.int32, L)

        def prime(ci, gbuf, sg):
            pltpu.async_copy(h_hbm.at[srcv.at[ci]], gbuf, sg)

        def waitbuf(gbuf, sg):
            pltpu.make_async_copy(h_hbm.at[pl.ds(0, CH)], gbuf, sg).wait()

        def compute(ci, gbuf, rows, ssem):
            pltpu.make_async_copy(rows, table.at[dstv.at[ci]], ssem).wait()
            civ = jnp.full((L,), ci, jnp.int32)
            for g in range(CH // L):
                e0 = g * L
                dst16 = plsc.load_gather(dstv, [civ, iota + e0])
                src16 = plsc.load_gather(srcv, [civ, iota + e0])
                dlg = plsc.load_gather(dl_v, [dst16])
                slg = plsc.load_gather(sl_v, [src16])
                l0 = dlg + slg
                pvv = jnp.exp(jnp.maximum(l0, 0.01 * l0))
                for k in range(L):
                    e = e0 + k
                    pspl = jnp.full((L,), pvv[k], jnp.float32)
                    for c in range(4):
                        rows[e, pl.ds(c * L, L)] = (
                            gbuf[e, pl.ds(c * L, L)] * pspl)
                    rows[e, pl.ds(F, L)] = pspl
            pltpu.async_copy(rows, table.at[dstv.at[ci]], ssem, add=True)

        pltpu.sync_copy(z_hbm.at[pl.ds(0, CH)], rows_a)
        pltpu.sync_copy(z_hbm.at[pl.ds(0, CH)], rows_b)
        pltpu.async_copy(rows_a, table.at[dstv.at[0]], ssa, add=True)
        pltpu.async_copy(rows_b, table.at[dstv.at[0]], ssb, add=True)
        prime(0, ga, sga)

        def pair_body(t, carry):
            c0 = 2 * t
            prime(c0 + 1, gb, sgb)
            waitbuf(ga, sga)
            compute(c0, ga, rows_a, ssa)
            prime(c0 + 2, ga, sga)
            waitbuf(gb, sgb)
            compute(c0 + 1, gb, rows_b, ssb)
            return carry

        lax.fori_loop(0, (NCHUNK - 1) // 2, pair_body, 0)
        waitbuf(ga, sga)
        compute(NCHUNK - 1, ga, rows_a, ssa)
        pltpu.make_async_copy(rows_a, table.at[dstv.at[0]], ssa).wait()
        pltpu.make_async_copy(rows_b, table.at[dstv.at[0]], ssb).wait()
        plsc.subcore_barrier()
        pltpu.sync_copy(table.at[pl.ds(sid * RZ_STRIDE, RZ_SIZE)],
                        out_hbm.at[cid, pl.ds(sid * RZ_STRIDE, RZ_SIZE)])

    return body(src3, dst3, h, dl, sl, zeros)


def kernel(node_feats, edge_feats, edge_index, params):
    p = params
    src3 = edge_index[0].astype(jnp.int32).reshape(NW, NCHUNK, CH)
    dst3 = edge_index[1].astype(jnp.int32).reshape(NW, NCHUNK, CH)

    pnb = p['pn_b'].reshape(1, F)
    w1a = p['pe1_W'][:128]
    w1b = p['pe1_W'][128:]
    b1 = p['pe1_b'].reshape(1, F)
    w2a = p['pe2_W'][:F]
    w2b = p['pe2_W'][F:, 0]
    b2 = p['pe2_b'].reshape(1, 1)

    hv, nfa, d1 = _tc_prep(node_feats, p['pn_W'], pnb, w1a, b1, w2a, b2)
    ef3 = edge_feats.reshape(NW, NCHUNK, CH * 4)

    zeros = jnp.zeros((N, RW), jnp.float32)
    part1 = _sc_round1(src3, dst3, nfa, ef3, d1.reshape(N), w2b,
                       w1b.reshape(4 * F), zeros)

    g1 = p['gru1']
    lp = p['layers'][0]
    h, dl, sl = _tc_mid(
        part1, hv, p['g1_et_W'], p['g1_et_b'].reshape(1, F),
        g1['W_ih'].T, g1['W_hh'].T,
        g1['b_ih'].reshape(1, 3 * F), g1['b_hh'].reshape(1, 3 * F),
        lp['pe_W'][:F], lp['pe_b'].reshape(1, 1), lp['pe_W'][F:])

    part2 = _sc_round2(src3, dst3, h, dl.reshape(N), sl.reshape(N), zeros)

    lg = lp['gru']
    fargs = [lp['pn_W'], lp['pn_b'].reshape(1, F),
             lg['W_ih'].T, lg['W_hh'].T,
             lg['b_ih'].reshape(1, 3 * F), lg['b_hh'].reshape(1, 3 * F)]
    for rp in p['readouts']:
        rg = rp['gru']
        fargs += [rp['cl_W'][:F], rp['cl_W'][F:], rp['cl_b'].reshape(1, 1),
                  rp['pn_W'], rp['pn_b'].reshape(1, F),
                  rg['W_ih'].T, rg['W_hh'].T,
                  rg['b_ih'].reshape(1, 3 * F), rg['b_hh'].reshape(1, 3 * F)]
    return _tc_final(part2, h, fargs)


# speedup vs baseline: 1.4056x; 1.4056x over previous
import functools

import jax
import jax.numpy as jnp
from jax import lax
from jax.experimental import pallas as pl
from jax.experimental.pallas import tpu as pltpu
from jax.experimental.pallas import tpu_sc as plsc

N = 10000
E = 320000
F = 64
NC, NS, L = 2, 16, 16
NW = NC * NS
E_W = E // NW
CH = 80
NCHUNK = E_W // CH
RW = 80
RZ_STRIDE = 624
RZ_SIZE = 640
GR = 8


def _leaky(x):
    return jnp.maximum(x, 0.01 * x)


def _elu(x):
    return jnp.where(x > 0, x, jnp.exp(jnp.minimum(x, 0.0)) - 1.0)


def _gru_tc(ctx, h, wih_t, whh_t, bih, bhh):
    gi = jnp.dot(ctx, wih_t) + bih
    gh = jnp.dot(h, whh_t) + bhh
    r = jax.nn.sigmoid(gi[:, 0:F] + gh[:, 0:F])
    z = jax.nn.sigmoid(gi[:, F:2 * F] + gh[:, F:2 * F])
    nn_ = jnp.tanh(gi[:, 2 * F:] + r * gh[:, 2 * F:])
    return (1.0 - z) * nn_ + z * h


def _treesum(vs):
    while len(vs) > 1:
        vs = [a + b for a, b in zip(vs[0::2], vs[1::2])] + (
            [vs[-1]] if len(vs) % 2 else [])
    return vs[0]


def _tc_prep_body(nf_ref, pnw_ref, pnb_ref, w1a_ref, b1_ref, w2a_ref, b2_ref,
                  hv_ref, nfa_ref, d1_ref):
    nf = nf_ref[...]
    hv = _leaky(jnp.dot(nf, pnw_ref[...]) + pnb_ref[...])
    hv_ref[...] = hv
    nfa_ref[...] = jnp.dot(nf, w1a_ref[...]) + b1_ref[...]
    d1_ref[...] = jnp.dot(hv, w2a_ref[...]) + b2_ref[...]


def _tc_prep(nf, pnw, pnb, w1a, b1, w2a, b2):
    return pl.pallas_call(
        _tc_prep_body,
        out_shape=(
            jax.ShapeDtypeStruct((N, F), jnp.float32),
            jax.ShapeDtypeStruct((N, F), jnp.float32),
            jax.ShapeDtypeStruct((N, 1), jnp.float32),
        ),
    )(nf, pnw, pnb, w1a, b1, w2a, b2)


def _tc_mid_body(part_ref, hv_ref, g1w_ref, g1b_ref, wih_ref, whh_ref,
                 bih_ref, bhh_ref, wd_ref, pb_ref, ws_ref,
                 h_ref, dl_ref, sl_ref):
    pa = part_ref[0]
    pb2 = part_ref[1]
    s = pa[:, F:F + 1] + pb2[:, F:F + 1]
    wsum = pa[:, 0:F] + pb2[:, 0:F]
    pos = s > 0
    inv = jnp.where(pos, 1.0 / jnp.where(pos, s, 1.0), 0.0)
    ind = pos.astype(jnp.float32)
    ctx = _elu(jnp.dot(wsum * inv, g1w_ref[...]) + ind * g1b_ref[...])
    hv = hv_ref[...]
    h = jax.nn.relu(_gru_tc(ctx, hv, wih_ref[...], whh_ref[...],
                            bih_ref[...], bhh_ref[...]))
    h_ref[...] = h
    dl_ref[...] = jnp.dot(h, wd_ref[...]) + pb_ref[...]
    sl_ref[...] = jnp.dot(h, ws_ref[...])


def _tc_mid(part1, hv, g1w, g1b, wih, whh, bih, bhh, wd, pb, ws):
    return pl.pallas_call(
        _tc_mid_body,
        out_shape=(
            jax.ShapeDtypeStruct((N, F), jnp.float32),
            jax.ShapeDtypeStruct((N, 1), jnp.float32),
            jax.ShapeDtypeStruct((N, 1), jnp.float32),
        ),
    )(part1, hv, g1w, g1b, wih, whh, bih, bhh, wd, pb, ws)


def _tc_final_body(part_ref, h_ref, pnw_ref, pnb_ref, wih_ref, whh_ref,
                   bih_ref, bhh_ref,
                   cg0_ref, ch0_ref, cb0_ref, rpw0_ref, rpb0_ref,
                   rwih0_ref, rwhh0_ref, rbih0_ref, rbhh0_ref,
                   cg1_ref, ch1_ref, cb1_ref, rpw1_ref, rpb1_ref,
                   rwih1_ref, rwhh1_ref, rbih1_ref, rbhh1_ref,
                   g_ref):
    pa = part_ref[0]
    pb2 = part_ref[1]
    s = pa[:, F:F + 1] + pb2[:, F:F + 1]
    wsum = pa[:, 0:F] + pb2[:, 0:F]
    pos = s > 0
    inv = jnp.where(pos, 1.0 / jnp.where(pos, s, 1.0), 0.0)
    ind = pos.astype(jnp.float32)
    ctx = _elu(jnp.dot(wsum * inv, pnw_ref[...]) + ind * pnb_ref[...])
    h_prev = h_ref[...]
    h = jax.nn.relu(_gru_tc(ctx, h_prev, wih_ref[...], whh_ref[...],
                            bih_ref[...], bhh_ref[...]))
    g = jnp.sum(h, axis=0, keepdims=True)
    ro = [(cg0_ref, ch0_ref, cb0_ref, rpw0_ref, rpb0_ref,
           rwih0_ref, rwhh0_ref, rbih0_ref, rbhh0_ref),
          (cg1_ref, ch1_ref, cb1_ref, rpw1_ref, rpb1_ref,
           rwih1_ref, rwhh1_ref, rbih1_ref, rbhh1_ref)]
    for (cg, chh, cb, rpw, rpb, rwih, rwhh, rbih, rbhh) in ro:
        zc = jnp.dot(jax.nn.relu(g), cg[...]) + cb[...]
        z = _leaky(zc + jnp.dot(h, chh[...]))
        zm = jnp.max(z, axis=0, keepdims=True)
        ez = jnp.exp(z - zm)
        aa = ez / jnp.sum(ez, axis=0, keepdims=True)
        hvp = jnp.dot(h, rpw[...]) + rpb[...]
        g_repr = _elu(jnp.sum(aa * hvp, axis=0, keepdims=True))
        g = _gru_tc(jax.nn.relu(g_repr), g, rwih[...], rwhh[...],
                    rbih[...], rbhh[...])
    g_ref[...] = g


def _tc_final(part2, h, args):
    return pl.pallas_call(
        _tc_final_body,
        out_shape=jax.ShapeDtypeStruct((1, F), jnp.float32),
    )(part2, h, *args)


def _sc_round1(src3, dst3, nfa, ef3, d1, w2b, w1b, zeros):
    mesh = plsc.VectorSubcoreMesh(core_axis_name="c", subcore_axis_name="s",
                                  num_cores=NC, num_subcores=NS)

    @functools.partial(
        pl.kernel,
        out_type=jax.ShapeDtypeStruct((NC, N, RW), jnp.float32),
        mesh=mesh,
        compiler_params=pltpu.CompilerParams(needs_layout_passes=False,
                                             use_tc_tiling_on_sc=False),
        scratch_types=[
            pltpu.VMEM((NCHUNK, CH), jnp.int32),
            pltpu.VMEM((NCHUNK, CH), jnp.int32),
            pltpu.VMEM((CH, F), jnp.float32),
            pltpu.VMEM((CH, F), jnp.float32),
            pltpu.VMEM((4, CH), jnp.float32),
            pltpu.VMEM((4, CH), jnp.float32),
            pltpu.VMEM((CH, F), jnp.float32),
            pltpu.VMEM((CH, RW), jnp.float32),
            pltpu.VMEM((CH, RW), jnp.float32),
            pltpu.VMEM((L, L), jnp.float32),
            pltpu.VMEM((N,), jnp.float32),
            pltpu.VMEM((F,), jnp.float32),
            pltpu.VMEM((4 * F,), jnp.float32),
            pltpu.VMEM_SHARED((N, RW), jnp.float32),
            pltpu.SemaphoreType.DMA,
            pltpu.SemaphoreType.DMA,
            pltpu.SemaphoreType.DMA,
            pltpu.SemaphoreType.DMA,
            pltpu.SemaphoreType.DMA,
            pltpu.SemaphoreType.DMA,
        ],
    )
    def body(src_hbm, dst_hbm, nfa_hbm, ef_hbm, d1_hbm, w2b_hbm, w1b_hbm,
             z_hbm, out_hbm, srcv, dstv, ga, gb, ea, eb, epjb, rows_a,
             rows_b, accv, d1_v, w2b_v, w1b_v, table, sga, sgb, sea, seb,
             ssa, ssb):
        cid = lax.axis_index("c")
        sid = lax.axis_index("s")
        wid = sid * NC + cid
        pltpu.sync_copy(d1_hbm, d1_v)
        pltpu.sync_copy(w2b_hbm, w2b_v)
        pltpu.sync_copy(w1b_hbm, w1b_v)
        pltpu.sync_copy(src_hbm.at[wid], srcv)
        pltpu.sync_copy(dst_hbm.at[wid], dstv)
        pltpu.sync_copy(z_hbm.at[pl.ds(sid * RZ_STRIDE, RZ_SIZE)],
                        table.at[pl.ds(sid * RZ_STRIDE, RZ_SIZE)])
        plsc.subcore_barrier()
        w2 = [w2b_v[pl.ds(c * L, L)] for c in range(4)]
        w1 = [[w1b_v[pl.ds(k * F + c * L, L)] for c in range(4)]
              for k in range(4)]
        iota = lax.iota(jnp.int32, L)

        def prime(ci, gbuf, ebuf, sg, se):
            pltpu.async_copy(nfa_hbm.at[srcv.at[ci]], gbuf, sg)
            pltpu.async_copy(ef_hbm.at[wid, ci], ebuf, se)

        def waitbuf(gbuf, ebuf, sg, se):
            pltpu.make_async_copy(nfa_hbm.at[pl.ds(0, CH)], gbuf, sg).wait()
            pltpu.make_async_copy(ef_hbm.at[0, 0], ebuf, se).wait()

        def expand_epj(g, ebuf):
            e0 = g * L
            efk = [ebuf[kk, pl.ds(e0, L)] for kk in range(4)]
            for k in range(L):
                e = e0 + k
                sp = [jnp.full((L,), efk[kk][k], jnp.float32)
                      for kk in range(4)]
                for c in range(4):
                    epjb[e, pl.ds(c * L, L)] = (
                        (sp[0] * w1[0][c] + sp[1] * w1[1][c])
                        + (sp[2] * w1[2][c] + sp[3] * w1[3][c]))
            return 0

        def compute(ci, gbuf, ebuf, rows, ssem):
            pltpu.make_async_copy(rows, table.at[dstv.at[ci]], ssem).wait()
            lax.fori_loop(0, CH // L, lambda g, c2: expand_epj(g, ebuf), 0)
            civ = jnp.full((L,), ci, jnp.int32)
            for g in range(CH // L):
                e0 = g * L
                dst16 = plsc.load_gather(dstv, [civ, iota + e0])
                d1g = plsc.load_gather(d1_v, [dst16])
                for k in range(L):
                    e = e0 + k
                    ts = []
                    for c in range(4):
                        x = (gbuf[e, pl.ds(c * L, L)]
                             + epjb[e, pl.ds(c * L, L)])
                        h1 = jnp.maximum(x, 0.01 * x)
                        rows[e, pl.ds(c * L, L)] = h1
                        ts.append(h1 * w2[c])
                    accv[k] = (ts[0] + ts[1]) + (ts[2] + ts[3])
                cols = [plsc.load_gather(accv,
                                         [iota, jnp.full((L,), l, jnp.int32)])
                        for l in range(L)]
                l0 = d1g + _treesum(cols)
                pvv = jnp.exp(jnp.maximum(l0, 0.01 * l0))
                for k in range(L):
                    e = e0 + k
                    pspl = jnp.full((L,), pvv[k], jnp.float32)
                    for c in range(4):
                        rows[e, pl.ds(c * L, L)] = (
                            rows[e, pl.ds(c * L, L)] * pspl)
                    rows[e, pl.ds(F, L)] = pspl
            pltpu.async_copy(rows, table.at[dstv.at[ci]], ssem, add=True)

        pltpu.sync_copy(z_hbm.at[pl.ds(0, CH)], rows_a)
        pltpu.sync_copy(z_hbm.at[pl.ds(0, CH)], rows_b)
        pltpu.async_copy(rows_a, table.at[dstv.at[0]], ssa, add=True)
        pltpu.async_copy(rows_b, table.at[dstv.at[0]], ssb, add=True)
        prime(0, ga, ea, sga, sea)

        def pair_body(t, carry):
            c0 = 2 * t
            prime(c0 + 1, gb, eb, sgb, seb)
            waitbuf(ga, ea, sga, sea)
            compute(c0, ga, ea, rows_a, ssa)
            prime(c0 + 2, ga, ea, sga, sea)
            waitbuf(gb, eb, sgb, seb)
            compute(c0 + 1, gb, eb, rows_b, ssb)
            return carry

        lax.fori_loop(0, (NCHUNK - 1) // 2, pair_body, 0)
        waitbuf(ga, ea, sga, sea)
        compute(NCHUNK - 1, ga, ea, rows_a, ssa)
        pltpu.make_async_copy(rows_a, table.at[dstv.at[0]], ssa).wait()
        pltpu.make_async_copy(rows_b, table.at[dstv.at[0]], ssb).wait()
        plsc.subcore_barrier()
        pltpu.sync_copy(table.at[pl.ds(sid * RZ_STRIDE, RZ_SIZE)],
                        out_hbm.at[cid, pl.ds(sid * RZ_STRIDE, RZ_SIZE)])

    return body(src3, dst3, nfa, ef3, d1, w2b, w1b, zeros)


def _sc_round2(src3, dst3, h, dl, sl, zeros):
    mesh = plsc.VectorSubcoreMesh(core_axis_name="c", subcore_axis_name="s",
                                  num_cores=NC, num_subcores=NS)

    @functools.partial(
        pl.kernel,
        out_type=jax.ShapeDtypeStruct((NC, N, RW), jnp.float32),
        mesh=mesh,
        compiler_params=pltpu.CompilerParams(needs_layout_passes=False,
                                             use_tc_tiling_on_sc=False),
        scratch_types=[
            pltpu.VMEM((NCHUNK, CH), jnp.int32),
            pltpu.VMEM((NCHUNK, CH), jnp.int32),
            pltpu.VMEM((CH, F), jnp.float32),
            pltpu.VMEM((CH, F), jnp.float32),
            pltpu.VMEM((CH, RW), jnp.float32),
            pltpu.VMEM((CH, RW), jnp.float32),
            pltpu.VMEM((N,), jnp.float32),
            pltpu.VMEM((N,), jnp.float32),
            pltpu.VMEM_SHARED((N, RW), jnp.float32),
            pltpu.SemaphoreType.DMA,
            pltpu.SemaphoreType.DMA,
            pltpu.SemaphoreType.DMA,
            pltpu.SemaphoreType.DMA,
        ],
    )
    def body(src_hbm, dst_hbm, h_hbm, dl_hbm, sl_hbm, z_hbm, out_hbm,
             srcv, dstv, ga, gb, rows_a, rows_b, dl_v, sl_v, table,
             sga, sgb, ssa, ssb):
        cid = lax.axis_index("c")
        sid = lax.axis_index("s")
        wid = sid * NC + cid
        pltpu.sync_copy(dl_hbm, dl_v)
        pltpu.sync_copy(sl_hbm, sl_v)
        pltpu.sync_copy(src_hbm.at[wid], srcv)
        pltpu.sync_copy(dst_hbm.at[wid], dstv)
        pltpu.sync_copy(z_hbm.at[pl.ds(sid * RZ_STRIDE, RZ_SIZE)],
                        table.at[pl.ds(sid * RZ_STRIDE, RZ_SIZE)])
        plsc.subcore_barrier()
        iota = lax.iota(jnp.int32, L)

        def prime(ci, gbuf, sg):
            pltpu.async_copy(h_hbm.at[srcv.at[ci]], gbuf, sg)

        def waitbuf(gbuf, sg):
            pltpu.make_async_copy(h_hbm.at[pl.ds(0, CH)], gbuf, sg).wait()

        def compute(ci, gbuf, rows, ssem):
            pltpu.make_async_copy(rows, table.at[dstv.at[ci]], ssem).wait()
            civ = jnp.full((L,), ci, jnp.int32)
            for g in range(CH // L):
                e0 = g * L
                dst16 = plsc.load_gather(dstv, [civ, iota + e0])
                src16 = plsc.load_gather(srcv, [civ, iota + e0])
                dlg = plsc.load_gather(dl_v, [dst16])
                slg = plsc.load_gather(sl_v, [src16])
                l0 = dlg + slg
                pvv = jnp.exp(jnp.maximum(l0, 0.01 * l0))
                for k in range(L):
                    e = e0 + k
                    pspl = jnp.full((L,), pvv[k], jnp.float32)
                    for c in range(4):
                        rows[e, pl.ds(c * L, L)] = (
                            gbuf[e, pl.ds(c * L, L)] * pspl)
                    rows[e, pl.ds(F, L)] = pspl
            pltpu.async_copy(rows, table.at[dstv.at[ci]], ssem, add=True)

        pltpu.sync_copy(z_hbm.at[pl.ds(0, CH)], rows_a)
        pltpu.sync_copy(z_hbm.at[pl.ds(0, CH)], rows_b)
        pltpu.async_copy(rows_a, table.at[dstv.at[0]], ssa, add=True)
        pltpu.async_copy(rows_b, table.at[dstv.at[0]], ssb, add=True)
        prime(0, ga, sga)

        def pair_body(t, carry):
            c0 = 2 * t
            prime(c0 + 1, gb, sgb)
            waitbuf(ga, sga)
            compute(c0, ga, rows_a, ssa)
            prime(c0 + 2, ga, sga)
            waitbuf(gb, sgb)
            compute(c0 + 1, gb, rows_b, ssb)
            return carry

        lax.fori_loop(0, (NCHUNK - 1) // 2, pair_body, 0)
        waitbuf(ga, sga)
        compute(NCHUNK - 1, ga, rows_a, ssa)
        pltpu.make_async_copy(rows_a, table.at[dstv.at[0]], ssa).wait()
        pltpu.make_async_copy(rows_b, table.at[dstv.at[0]], ssb).wait()
        plsc.subcore_barrier()
        pltpu.sync_copy(table.at[pl.ds(sid * RZ_STRIDE, RZ_SIZE)],
                        out_hbm.at[cid, pl.ds(sid * RZ_STRIDE, RZ_SIZE)])

    return body(src3, dst3, h, dl, sl, zeros)


def kernel(node_feats, edge_feats, edge_index, params):
    p = params
    src3 = edge_index[0].astype(jnp.int32).reshape(NW, NCHUNK, CH)
    dst3 = edge_index[1].astype(jnp.int32).reshape(NW, NCHUNK, CH)

    pnb = p['pn_b'].reshape(1, F)
    w1a = p['pe1_W'][:128]
    w1b = p['pe1_W'][128:]
    b1 = p['pe1_b'].reshape(1, F)
    w2a = p['pe2_W'][:F]
    w2b = p['pe2_W'][F:, 0]
    b2 = p['pe2_b'].reshape(1, 1)

    hv, nfa, d1 = _tc_prep(node_feats, p['pn_W'], pnb, w1a, b1, w2a, b2)
    ef3 = edge_feats.T.reshape(4, NW, NCHUNK, CH).transpose(1, 2, 0, 3)

    zeros = jnp.zeros((N, RW), jnp.float32)
    part1 = _sc_round1(src3, dst3, nfa, ef3, d1.reshape(N), w2b,
                       w1b.reshape(4 * F), zeros)

    g1 = p['gru1']
    lp = p['layers'][0]
    h, dl, sl = _tc_mid(
        part1, hv, p['g1_et_W'], p['g1_et_b'].reshape(1, F),
        g1['W_ih'].T, g1['W_hh'].T,
        g1['b_ih'].reshape(1, 3 * F), g1['b_hh'].reshape(1, 3 * F),
        lp['pe_W'][:F], lp['pe_b'].reshape(1, 1), lp['pe_W'][F:])

    part2 = _sc_round2(src3, dst3, h, dl.reshape(N), sl.reshape(N), zeros)

    lg = lp['gru']
    fargs = [lp['pn_W'], lp['pn_b'].reshape(1, F),
             lg['W_ih'].T, lg['W_hh'].T,
             lg['b_ih'].reshape(1, 3 * F), lg['b_hh'].reshape(1, 3 * F)]
    for rp in p['readouts']:
        rg = rp['gru']
        fargs += [rp['cl_W'][:F], rp['cl_W'][F:], rp['cl_b'].reshape(1, 1),
                  rp['pn_W'], rp['pn_b'].reshape(1, F),
                  rg['W_ih'].T, rg['W_hh'].T,
                  rg['b_ih'].reshape(1, 3 * F), rg['b_hh'].reshape(1, 3 * F)]
    return _tc_final(part2, h, fargs)
